# baseline (speedup 1.0000x reference)
import functools

import jax
import jax.numpy as jnp
from jax import lax
from jax.experimental import pallas as pl
from jax.experimental.pallas import tpu as pltpu
from jax.experimental.pallas import tpu_sc as plsc

GAMMA = 1.0
LAM = 1.0
RHO = 1.0
N_ADMM = 10
N_JACOBI = 5
N_BISECT = 26

L = 16
U = 4


def _build(B, N, E, interpret=False):
    TPG = 4
    EW = E // TPG
    GPC = B // 2

    mesh = plsc.VectorSubcoreMesh(core_axis_name="c", subcore_axis_name="s",
                                  num_cores=2, num_subcores=16)

    @functools.partial(
        pl.kernel,
        out_type=jax.ShapeDtypeStruct((B * N,), jnp.float32),
        mesh=mesh,
        scratch_types=[
            pltpu.VMEM((EW,), jnp.int32),
            pltpu.VMEM((EW,), jnp.int32),
            pltpu.VMEM((EW,), jnp.float32),
            pltpu.VMEM((N,), jnp.float32),
            pltpu.VMEM((N,), jnp.float32),
            pltpu.VMEM((N,), jnp.float32),
            pltpu.VMEM((N,), jnp.float32),
            pltpu.VMEM((N,), jnp.float32),
            pltpu.VMEM((N,), jnp.float32),
            pltpu.VMEM((N,), jnp.float32),
            pltpu.VMEM((N,), jnp.float32),
            pltpu.VMEM_SHARED((2, GPC, TPG, N), jnp.float32),
            pltpu.SemaphoreType.DMA,
        ],
        compiler_params=pltpu.CompilerParams(needs_layout_passes=False),
        interpret=interpret,
    )
    def gfusedmax_kernel(x_hbm, src_hbm, dst_hbm, out_hbm,
                         src_v, dst_v, w_v, x_v, z_v, b_v, invd_v,
                         acc_v, t1_v, t2_v, t3_v, shared, sem):
        c = lax.axis_index("c")
        s = lax.axis_index("s")
        gl = s // TPG
        q = s % TPG
        g = c * GPC + gl

        NCH = N // (L * U)
        ECH = EW // (L * U)
        zeros = jnp.zeros((L,), jnp.float32)
        ones = jnp.ones((L,), jnp.float32)

        pltpu.sync_copy(x_hbm.at[pl.ds(g * N, N)], x_v)
        pltpu.sync_copy(src_hbm.at[pl.ds(g * E + q * EW, EW)], src_v)
        pltpu.sync_copy(dst_hbm.at[pl.ds(g * E + q * EW, EW)], dst_v)

        def combine(parity):
            pltpu.sync_copy(acc_v, shared.at[parity, gl, q])
            plsc.subcore_barrier()
            d1 = pltpu.async_copy(
                shared.at[parity, gl, lax.rem(q + 1, TPG)], t1_v, sem)
            d2 = pltpu.async_copy(
                shared.at[parity, gl, lax.rem(q + 2, TPG)], t2_v, sem)
            d3 = pltpu.async_copy(
                shared.at[parity, gl, lax.rem(q + 3, TPG)], t3_v, sem)
            d1.wait()
            d2.wait()
            d3.wait()

            @plsc.parallel_loop(0, N // L, unroll=U)
            def abody(i):
                ds = pl.ds(i * L, L)
                acc_v[ds] = (acc_v[ds] + t1_v[ds]) + (t2_v[ds] + t3_v[ds])

        @plsc.parallel_loop(0, N // L, unroll=U)
        def zinit_body(i):
            ds = pl.ds(i * L, L)
            acc_v[ds] = zeros
            z_v[ds] = x_v[ds]

        @plsc.parallel_loop(0, EW // L, unroll=U)
        def winit_body(i):
            ds = pl.ds(i * L, L)
            w_v[ds] = zeros

        @plsc.parallel_loop(0, EW // L, unroll=U)
        def deg_body(i):
            e = pl.ds(i * L, L)
            plsc.addupdate_scatter(acc_v, [src_v[e]], ones)
            plsc.addupdate_scatter(acc_v, [dst_v[e]], ones)
        combine(0)

        @plsc.parallel_loop(0, N // L, unroll=U)
        def invd_body(i):
            ds = pl.ds(i * L, L)
            av = acc_v[ds]
            invd_v[ds] = 1.0 / (1.0 + RHO * av)
            acc_v[ds] = av * 0.0

        def admm_body(it, tt):
            combine(1)

            @plsc.parallel_loop(0, N // L, unroll=U)
            def b_body(i):
                ds = pl.ds(i * L, L)
                av = acc_v[ds]
                b_v[ds] = x_v[ds] + RHO * av
                acc_v[ds] = av * 0.0

            def jac_body(jj, t):
                with jax.named_scope("az_pass"):
                    @plsc.parallel_loop(0, EW // L, unroll=U)
                    def az_body(i):
                        e = pl.ds(i * L, L)
                        sv = src_v[e]
                        dv = dst_v[e]
                        zs = plsc.load_gather(z_v, [sv])
                        zd = plsc.load_gather(z_v, [dv])
                        plsc.addupdate_scatter(acc_v, [sv], zd)
                        plsc.addupdate_scatter(acc_v, [dv], zs)
                with jax.named_scope("combine"):
                    combine(lax.rem(jj, 2))

                with jax.named_scope("z_pass"):
                    @plsc.parallel_loop(0, N // L, unroll=U)
                    def z_body(i):
                        ds = pl.ds(i * L, L)
                        av = acc_v[ds]
                        z_v[ds] = (b_v[ds] + RHO * av) * invd_v[ds]
                        acc_v[ds] = av * 0.0
                return t
            lax.fori_loop(0, N_JACOBI, jac_body, 0)

            with jax.named_scope("uw_pass"):
                return _uw(tt)

        def _uw(tt):
            @plsc.parallel_loop(0, EW // L, unroll=U)
            def uw_body(i):
                e = pl.ds(i * L, L)
                sv = src_v[e]
                dv = dst_v[e]
                zs = plsc.load_gather(z_v, [sv])
                zd = plsc.load_gather(z_v, [dv])
                tv = (zs - zd) + w_v[e]
                un = tv - jnp.minimum(jnp.maximum(tv, -(LAM / RHO)),
                                      (LAM / RHO))
                wn = tv - un
                w_v[e] = wn
                rv = un - wn
                plsc.addupdate_scatter(acc_v, [sv], rv)
                plsc.addupdate_scatter(acc_v, [dv], -rv)
            return tt
        lax.fori_loop(0, N_ADMM, admm_body, 0)

        inv_gamma = 1.0 / GAMMA

        def max_body(i, mvs):
            return tuple(
                jnp.maximum(mvs[j], z_v[pl.ds((i * U + j) * L, L)] * inv_gamma)
                for j in range(U))
        ninf = jnp.full((L,), -jnp.inf, jnp.float32)
        mvs = lax.fori_loop(0, NCH, max_body, (ninf,) * U)
        zmax = jnp.max(jnp.maximum(jnp.maximum(mvs[0], mvs[1]),
                                   jnp.maximum(mvs[2], mvs[3])))

        def bisect_body(_, carry):
            lo, hi = carry
            tau = 0.5 * (lo + hi)

            def s_body(i, svs):
                return tuple(
                    svs[j] + jnp.maximum(
                        z_v[pl.ds((i * U + j) * L, L)] * inv_gamma - tau, 0.0)
                    for j in range(U))
            svs = lax.fori_loop(0, NCH, s_body, (zeros,) * U)
            stot = jnp.sum((svs[0] + svs[1]) + (svs[2] + svs[3]))
            big = stot >= 1.0
            lo = jnp.where(big, tau, lo)
            hi = jnp.where(big, hi, tau)
            return (lo, hi)
        lo, hi = lax.fori_loop(0, N_BISECT, bisect_body,
                               (zmax - 1.0, zmax))
        tau0 = 0.5 * (lo + hi)

        def ref_body(i, carry):
            saccs, caccs = carry
            zcs = [z_v[pl.ds((i * U + j) * L, L)] * inv_gamma
                   for j in range(U)]
            msks = [zc > tau0 for zc in zcs]
            saccs = tuple(saccs[j] + jnp.where(msks[j], zcs[j], 0.0)
                          for j in range(U))
            caccs = tuple(caccs[j] + jnp.where(msks[j], 1.0, 0.0)
                          for j in range(U))
            return (saccs, caccs)
        svs2, cvs2 = lax.fori_loop(0, NCH, ref_body,
                                   ((zeros,) * U, (zeros,) * U))
        ssum = jnp.sum((svs2[0] + svs2[1]) + (svs2[2] + svs2[3]))
        kcnt = jnp.maximum(jnp.sum((cvs2[0] + cvs2[1]) + (cvs2[2] + cvs2[3])),
                           1.0)
        tau = (jnp.full((L,), ssum - 1.0, jnp.float32)
               / jnp.full((L,), kcnt, jnp.float32))

        NQ = N // TPG

        @plsc.parallel_loop(0, NQ // L, unroll=U)
        def out_body(i):
            zc = z_v[pl.ds(q * NQ + i * L, L)] * inv_gamma
            t1_v[pl.ds(i * L, L)] = jnp.maximum(zc - tau, 0.0)
        pltpu.sync_copy(t1_v.at[pl.ds(0, NQ)],
                        out_hbm.at[pl.ds(g * N + q * NQ, NQ)])

    return gfusedmax_kernel


def kernel(x, graph_size_list, edge_list):
    B = graph_size_list.shape[0]
    N = x.shape[0] // B
    E = edge_list.shape[1]
    src = edge_list[:, :, 0].reshape(-1)
    dst = edge_list[:, :, 1].reshape(-1)
    return _build(B, N, E)(x, src, dst)


# speedup vs baseline: 1.1446x; 1.1446x over previous
import functools

import jax
import jax.numpy as jnp
from jax import lax
from jax.experimental import pallas as pl
from jax.experimental.pallas import tpu as pltpu
from jax.experimental.pallas import tpu_sc as plsc

GAMMA = 1.0
LAM = 1.0
RHO = 1.0
N_ADMM = 10
N_JACOBI = 5
N_BISECT = 26

L = 16
U = 4


def _build(B, N, E, interpret=False):
    TPG = 4
    EW = E // TPG
    GPC = B // 2
    NQ = N // TPG

    mesh = plsc.VectorSubcoreMesh(core_axis_name="c", subcore_axis_name="s",
                                  num_cores=2, num_subcores=16)

    @functools.partial(
        pl.kernel,
        out_type=jax.ShapeDtypeStruct((B * N,), jnp.float32),
        mesh=mesh,
        scratch_types=[
            pltpu.VMEM((EW,), jnp.int32),
            pltpu.VMEM((EW,), jnp.int32),
            pltpu.VMEM((EW,), jnp.int32),
            pltpu.VMEM((EW,), jnp.float32),
            pltpu.VMEM((NQ,), jnp.float32),
            pltpu.VMEM((N,), jnp.float32),
            pltpu.VMEM((NQ,), jnp.float32),
            pltpu.VMEM((NQ,), jnp.float32),
            pltpu.VMEM((N,), jnp.float32),
            pltpu.VMEM((NQ,), jnp.float32),
            pltpu.VMEM((NQ,), jnp.float32),
            pltpu.VMEM((NQ,), jnp.float32),
            pltpu.VMEM((NQ,), jnp.float32),
            pltpu.VMEM((NQ,), jnp.float32),
            pltpu.VMEM_SHARED((2, GPC, TPG, N), jnp.float32),
            pltpu.VMEM_SHARED((GPC, N), jnp.float32),
            pltpu.SemaphoreType.DMA,
        ],
        compiler_params=pltpu.CompilerParams(needs_layout_passes=False),
        interpret=interpret,
    )
    def gfusedmax_kernel(x_hbm, src_hbm, dst_hbm, out_hbm,
                         src_v, dst_v, pk_v, w_v, xq_v, z_v, b_v, invd_v,
                         acc_v, t0_v, t1_v, t2_v, t3_v, zq_v,
                         shared, shz, sem):
        c = lax.axis_index("c")
        s = lax.axis_index("s")
        gl = s // TPG
        q = s % TPG
        g = c * GPC + gl

        NCH = N // (L * U)
        zeros = jnp.zeros((L,), jnp.float32)
        ones = jnp.ones((L,), jnp.float32)

        pltpu.sync_copy(x_hbm.at[pl.ds(g * N, N)], z_v)
        pltpu.sync_copy(x_hbm.at[pl.ds(g * N + q * NQ, NQ)], xq_v)
        pltpu.sync_copy(src_hbm.at[pl.ds(g * E + q * EW, EW)], src_v)
        pltpu.sync_copy(dst_hbm.at[pl.ds(g * E + q * EW, EW)], dst_v)

        def zero_acc():
            @plsc.parallel_loop(0, N // L, unroll=U)
            def zbody(i):
                acc_v[pl.ds(i * L, L)] = zeros

        def combine_quarters(parity):
            pltpu.sync_copy(acc_v, shared.at[parity, gl, q])
            zero_acc()
            plsc.subcore_barrier()
            ds_q = pl.ds(q * NQ, NQ)
            d0 = pltpu.async_copy(shared.at[parity, gl, q, ds_q], t0_v, sem)
            d1 = pltpu.async_copy(
                shared.at[parity, gl, lax.rem(q + 1, TPG), ds_q], t1_v, sem)
            d2 = pltpu.async_copy(
                shared.at[parity, gl, lax.rem(q + 2, TPG), ds_q], t2_v, sem)
            d3 = pltpu.async_copy(
                shared.at[parity, gl, lax.rem(q + 3, TPG), ds_q], t3_v, sem)
            d0.wait()
            d1.wait()
            d2.wait()
            d3.wait()

        def publish_z():
            pltpu.sync_copy(zq_v, shz.at[gl, pl.ds(q * NQ, NQ)])
            plsc.subcore_barrier()
            pltpu.sync_copy(shz.at[gl], z_v)

        @plsc.parallel_loop(0, EW // L, unroll=U)
        def winit_body(i):
            ds = pl.ds(i * L, L)
            w_v[ds] = zeros
            pk_v[ds] = jnp.bitwise_or(
                src_v[ds], jnp.left_shift(dst_v[ds], 12))

        zero_acc()

        @plsc.parallel_loop(0, EW // L, unroll=U)
        def deg_body(i):
            e = pl.ds(i * L, L)
            plsc.addupdate_scatter(acc_v, [src_v[e]], ones)
            plsc.addupdate_scatter(acc_v, [dst_v[e]], ones)
        combine_quarters(0)

        @plsc.parallel_loop(0, NQ // L, unroll=U)
        def invd_body(i):
            ds = pl.ds(i * L, L)
            av = (t0_v[ds] + t1_v[ds]) + (t2_v[ds] + t3_v[ds])
            invd_v[ds] = 1.0 / (1.0 + RHO * av)

        def admm_body(it, tt):
            combine_quarters(1)

            @plsc.parallel_loop(0, NQ // L, unroll=U)
            def b_body(i):
                ds = pl.ds(i * L, L)
                av = (t0_v[ds] + t1_v[ds]) + (t2_v[ds] + t3_v[ds])
                b_v[ds] = xq_v[ds] + RHO * av

            def jac_body(jj, t):
                @plsc.parallel_loop(0, EW // L, unroll=U)
                def az_body(i):
                    e = pl.ds(i * L, L)
                    pk = pk_v[e]
                    sv = jnp.bitwise_and(pk, 0xFFF)
                    dv = jnp.right_shift(pk, 12)
                    zs = plsc.load_gather(z_v, [sv])
                    zd = plsc.load_gather(z_v, [dv])
                    plsc.addupdate_scatter(acc_v, [sv], zd)
                    plsc.addupdate_scatter(acc_v, [dv], zs)
                combine_quarters(lax.rem(jj, 2))

                @plsc.parallel_loop(0, NQ // L, unroll=U)
                def z_body(i):
                    ds = pl.ds(i * L, L)
                    av = (t0_v[ds] + t1_v[ds]) + (t2_v[ds] + t3_v[ds])
                    zq_v[ds] = (b_v[ds] + RHO * av) * invd_v[ds]
                publish_z()
                return t
            lax.fori_loop(0, N_JACOBI, jac_body, 0)

            @plsc.parallel_loop(0, EW // L, unroll=U)
            def uw_body(i):
                e = pl.ds(i * L, L)
                pk = pk_v[e]
                sv = jnp.bitwise_and(pk, 0xFFF)
                dv = jnp.right_shift(pk, 12)
                zs = plsc.load_gather(z_v, [sv])
                zd = plsc.load_gather(z_v, [dv])
                tv = (zs - zd) + w_v[e]
                un = tv - jnp.minimum(jnp.maximum(tv, -(LAM / RHO)),
                                      (LAM / RHO))
                wn = tv - un
                w_v[e] = wn
                rv = un - wn
                plsc.addupdate_scatter(acc_v, [sv], rv)
                plsc.addupdate_scatter(acc_v, [dv], -rv)
            return tt
        lax.fori_loop(0, N_ADMM, admm_body, 0)

        inv_gamma = 1.0 / GAMMA

        def max_body(i, mvs):
            return tuple(
                jnp.maximum(mvs[j], z_v[pl.ds((i * U + j) * L, L)] * inv_gamma)
                for j in range(U))
        ninf = jnp.full((L,), -jnp.inf, jnp.float32)
        mvs = lax.fori_loop(0, NCH, max_body, (ninf,) * U)
        zmax = jnp.max(jnp.maximum(jnp.maximum(mvs[0], mvs[1]),
                                   jnp.maximum(mvs[2], mvs[3])))

        def bisect_body(_, carry):
            lo, hi = carry
            tau = 0.5 * (lo + hi)

            def s_body(i, svs):
                return tuple(
                    svs[j] + jnp.maximum(
                        z_v[pl.ds((i * U + j) * L, L)] * inv_gamma - tau, 0.0)
                    for j in range(U))
            svs = lax.fori_loop(0, NCH, s_body, (zeros,) * U)
            stot = jnp.sum((svs[0] + svs[1]) + (svs[2] + svs[3]))
            big = stot >= 1.0
            lo = jnp.where(big, tau, lo)
            hi = jnp.where(big, hi, tau)
            return (lo, hi)
        lo, hi = lax.fori_loop(0, N_BISECT, bisect_body,
                               (zmax - 1.0, zmax))
        tau0 = 0.5 * (lo + hi)

        def ref_body(i, carry):
            saccs, caccs = carry
            zcs = [z_v[pl.ds((i * U + j) * L, L)] * inv_gamma
                   for j in range(U)]
            msks = [zc > tau0 for zc in zcs]
            saccs = tuple(saccs[j] + jnp.where(msks[j], zcs[j], 0.0)
                          for j in range(U))
            caccs = tuple(caccs[j] + jnp.where(msks[j], 1.0, 0.0)
                          for j in range(U))
            return (saccs, caccs)
        svs2, cvs2 = lax.fori_loop(0, NCH, ref_body,
                                   ((zeros,) * U, (zeros,) * U))
        ssum = jnp.sum((svs2[0] + svs2[1]) + (svs2[2] + svs2[3]))
        kcnt = jnp.maximum(jnp.sum((cvs2[0] + cvs2[1]) + (cvs2[2] + cvs2[3])),
                           1.0)
        tau = (jnp.full((L,), ssum - 1.0, jnp.float32)
               / jnp.full((L,), kcnt, jnp.float32))

        @plsc.parallel_loop(0, NQ // L, unroll=U)
        def out_body(i):
            zc = z_v[pl.ds(q * NQ + i * L, L)] * inv_gamma
            zq_v[pl.ds(i * L, L)] = jnp.maximum(zc - tau, 0.0)
        pltpu.sync_copy(zq_v, out_hbm.at[pl.ds(g * N + q * NQ, NQ)])

    return gfusedmax_kernel


def kernel(x, graph_size_list, edge_list):
    B = graph_size_list.shape[0]
    N = x.shape[0] // B
    E = edge_list.shape[1]
    src = edge_list[:, :, 0].reshape(-1)
    dst = edge_list[:, :, 1].reshape(-1)
    return _build(B, N, E)(x, src, dst)
